# baseline (speedup 1.0000x reference)
import functools

import jax
import jax.numpy as jnp
from jax import lax
from jax.experimental import pallas as pl
from jax.experimental.pallas import tpu as pltpu
from jax.experimental.pallas import tpu_sc as plsc

M_TILE = 256


def _scores_body(x_ref, p_ref, xsq_ref, psq_ref, s_ref, idx_ref, iota_ref):
    @pl.when(pl.program_id(0) == 0)
    def _():
        iota_ref[...] = lax.broadcasted_iota(
            jnp.int32, iota_ref.shape, 1
        ).astype(jnp.float32)

    x = x_ref[...]
    cross2 = lax.dot_general(
        x + x, p_ref[...], (((1,), (1,)), ((), ())),
        preferred_element_type=jnp.float32,
    )
    scores = (cross2 - xsq_ref[...]) - psq_ref[...]
    s_ref[...] = scores
    m = jnp.max(scores, axis=1, keepdims=True)
    ii = iota_ref[0:1, :]
    big = jnp.float32(scores.shape[1])
    idx = jnp.min(jnp.where(scores == m, ii, big), axis=1)
    idx_ref[...] = idx.astype(jnp.int32).reshape(1, 1, M_TILE)


def _scores_and_argmax(xr, prototypes, x_sq, p_sq):
    M, d = xr.shape
    K = prototypes.shape[0]
    n_tiles = M // M_TILE
    scores, idx3 = pl.pallas_call(
        _scores_body,
        grid=(n_tiles,),
        in_specs=[
            pl.BlockSpec((M_TILE, d), lambda i: (i, 0)),
            pl.BlockSpec((K, d), lambda i: (0, 0)),
            pl.BlockSpec((M_TILE, 1), lambda i: (i, 0)),
            pl.BlockSpec((1, K), lambda i: (0, 0)),
        ],
        out_specs=[
            pl.BlockSpec((M_TILE, K), lambda i: (i, 0)),
            pl.BlockSpec((1, 1, M_TILE), lambda i: (i, 0, 0)),
        ],
        out_shape=[
            jax.ShapeDtypeStruct((M, K), jnp.float32),
            jax.ShapeDtypeStruct((n_tiles, 1, M_TILE), jnp.int32),
        ],
        scratch_shapes=[pltpu.VMEM((8, K), jnp.float32)],
    )(xr, prototypes, x_sq, p_sq)
    return scores, idx3.reshape(M)


def _make_sc_gather(V, D, B):
    info = plsc.get_sparse_core_info()
    NC, NS = info.num_cores, info.num_subcores
    NW = NC * NS
    assert D % info.num_lanes == 0 and B % (8 * NW) == 0
    b_per_w = B // NW
    mesh = plsc.VectorSubcoreMesh(core_axis_name="c", subcore_axis_name="s")

    @functools.partial(
        pl.kernel,
        mesh=mesh,
        out_type=jax.ShapeDtypeStruct((B, D), jnp.float32),
        scratch_types=[
            pltpu.VMEM((b_per_w,), jnp.int32),
            pltpu.VMEM((b_per_w, D), jnp.float32),
            pltpu.SemaphoreType.DMA,
        ],
    )
    def gather(table_hbm, idx_hbm, out_hbm, idx_v, rows_v, sem):
        wid = lax.axis_index("s") * NC + lax.axis_index("c")
        base = wid * b_per_w
        pltpu.sync_copy(idx_hbm.at[pl.ds(base, b_per_w)], idx_v)
        pltpu.async_copy(table_hbm.at[idx_v], rows_v, sem).wait()
        pltpu.sync_copy(rows_v, out_hbm.at[pl.ds(base, b_per_w)])

    return gather


def kernel(x, prototypes):
    B, N, d = x.shape
    K = prototypes.shape[0]
    M = B * N
    xr = x.reshape(M, d)
    x_sq = jnp.sum(xr * xr, axis=-1, keepdims=True)
    p_sq = jnp.sum(prototypes * prototypes, axis=-1)[None, :]
    scores_flat, idx = _scores_and_argmax(xr, prototypes, x_sq, p_sq)
    matched_flat = _make_sc_gather(K, d, M)(prototypes, idx)
    return matched_flat.reshape(B, N, d), scores_flat.reshape(B, N, K)


# speedup vs baseline: 1.0517x; 1.0517x over previous
import functools

import jax
import jax.numpy as jnp
from jax import lax
from jax.experimental import pallas as pl
from jax.experimental.pallas import tpu as pltpu
from jax.experimental.pallas import tpu_sc as plsc

M_TILE = 512


def _scores_body(x_ref, p_ref, xsq_ref, psq_ref, s_ref, idx_ref, iota_ref):
    @pl.when(pl.program_id(0) == 0)
    def _():
        iota_ref[...] = lax.broadcasted_iota(
            jnp.int32, iota_ref.shape, 1
        ).astype(jnp.float32)

    x = x_ref[...]
    cross2 = lax.dot_general(
        x + x, p_ref[...], (((1,), (1,)), ((), ())),
        preferred_element_type=jnp.float32,
    )
    scores = (cross2 - xsq_ref[...]) - psq_ref[...]
    s_ref[...] = scores
    m = jnp.max(scores, axis=1, keepdims=True)
    ii = iota_ref[0:1, :]
    big = jnp.float32(scores.shape[1])
    idx = jnp.min(jnp.where(scores == m, ii, big), axis=1)
    idx_ref[...] = idx.astype(jnp.int32).reshape(1, 1, M_TILE)


def _scores_and_argmax(xr, prototypes, x_sq, p_sq):
    M, d = xr.shape
    K = prototypes.shape[0]
    n_tiles = M // M_TILE
    scores, idx3 = pl.pallas_call(
        _scores_body,
        grid=(n_tiles,),
        in_specs=[
            pl.BlockSpec((M_TILE, d), lambda i: (i, 0)),
            pl.BlockSpec((K, d), lambda i: (0, 0)),
            pl.BlockSpec((M_TILE, 1), lambda i: (i, 0)),
            pl.BlockSpec((1, K), lambda i: (0, 0)),
        ],
        out_specs=[
            pl.BlockSpec((M_TILE, K), lambda i: (i, 0)),
            pl.BlockSpec((1, 1, M_TILE), lambda i: (i, 0, 0)),
        ],
        out_shape=[
            jax.ShapeDtypeStruct((M, K), jnp.float32),
            jax.ShapeDtypeStruct((n_tiles, 1, M_TILE), jnp.int32),
        ],
        scratch_shapes=[pltpu.VMEM((8, K), jnp.float32)],
    )(xr, prototypes, x_sq, p_sq)
    return scores, idx3.reshape(M)


def _make_sc_gather(V, D, B):
    info = plsc.get_sparse_core_info()
    NC, NS = info.num_cores, info.num_subcores
    NW = NC * NS
    assert D % info.num_lanes == 0 and B % (8 * NW) == 0
    b_per_w = B // NW
    mesh = plsc.VectorSubcoreMesh(core_axis_name="c", subcore_axis_name="s")

    @functools.partial(
        pl.kernel,
        mesh=mesh,
        out_type=jax.ShapeDtypeStruct((B, D), jnp.float32),
        scratch_types=[
            pltpu.VMEM((b_per_w,), jnp.int32),
            pltpu.VMEM((b_per_w, D), jnp.float32),
            pltpu.SemaphoreType.DMA,
        ],
    )
    def gather(table_hbm, idx_hbm, out_hbm, idx_v, rows_v, sem):
        wid = lax.axis_index("s") * NC + lax.axis_index("c")
        base = wid * b_per_w
        pltpu.sync_copy(idx_hbm.at[pl.ds(base, b_per_w)], idx_v)
        pltpu.async_copy(table_hbm.at[idx_v], rows_v, sem).wait()
        pltpu.sync_copy(rows_v, out_hbm.at[pl.ds(base, b_per_w)])

    return gather


def kernel(x, prototypes):
    B, N, d = x.shape
    K = prototypes.shape[0]
    M = B * N
    xr = x.reshape(M, d)
    x_sq = jnp.sum(xr * xr, axis=-1, keepdims=True)
    p_sq = jnp.sum(prototypes * prototypes, axis=-1)[None, :]
    scores_flat, idx = _scores_and_argmax(xr, prototypes, x_sq, p_sq)
    matched_flat = _make_sc_gather(K, d, M)(prototypes, idx)
    return matched_flat.reshape(B, N, d), scores_flat.reshape(B, N, K)
